# baseline (speedup 1.0000x reference)
import functools
import jax
import jax.numpy as jnp
from jax import lax
from jax.experimental import pallas as pl
from jax.experimental.pallas import tpu as pltpu
from jax.experimental.pallas import tpu_sc as plsc

_B = 16384
_NC = 2
_NS = 16
_NW = _NC * _NS
_BPW = _B // _NW
_NG = _BPW // 16
_RC = 4096
_NSTEPC = _B // _RC


def _fold_body(tcat_ref, w1_ref, b1_ref, out_ref):
    f = jnp.dot(tcat_ref[...], w1_ref[...], preferred_element_type=jnp.float32,
                precision=jax.lax.Precision.HIGHEST)
    out_ref[...] = f + 0.25 * b1_ref[...]


def _gather_body(f_hbm, m_hbm, d_hbm, w_hbm, h_hbm, out_hbm,
                 f_v, mi_v, di_v, wi_v, hi_v, h1_v):
    wid = lax.axis_index("s") * _NC + lax.axis_index("c")
    base = wid * _BPW
    pltpu.sync_copy(f_hbm, f_v)
    pltpu.sync_copy(m_hbm.at[pl.ds(base, _BPW)], mi_v)
    pltpu.sync_copy(d_hbm.at[pl.ds(base, _BPW)], di_v)
    pltpu.sync_copy(w_hbm.at[pl.ds(base, _BPW)], wi_v)
    pltpu.sync_copy(h_hbm.at[pl.ds(base, _BPW)], hi_v)

    def g_body(g, carry):
        o = g * 16
        mi = mi_v[pl.ds(o, 16)] * 64
        di = (di_v[pl.ds(o, 16)] + 13) * 64
        wi = (wi_v[pl.ds(o, 16)] + 45) * 64
        hi = (hi_v[pl.ds(o, 16)] + 52) * 64
        for k in range(64):
            e = (plsc.load_gather(f_v, [mi + k])
                 + plsc.load_gather(f_v, [di + k])
                 + plsc.load_gather(f_v, [wi + k])
                 + plsc.load_gather(f_v, [hi + k]))
            h1_v[k, pl.ds(o, 16)] = jnp.maximum(e, 0.0)
        return carry

    lax.fori_loop(0, _NG, g_body, 0)
    pltpu.sync_copy(h1_v, out_hbm.at[:, pl.ds(base, _BPW)])


def _mlp_body(h1_ref, w2_ref, b2_ref, w3_ref, b3_ref, out_ref):
    c00 = (((0,), (0,)), ((), ()))
    h1 = h1_ref[...]
    h2 = jax.lax.dot_general(w2_ref[...], h1, c00,
                             preferred_element_type=jnp.float32)
    h2 = jnp.maximum(h2 + b2_ref[...], 0.0)
    o = jax.lax.dot_general(w3_ref[...], h2, c00,
                            preferred_element_type=jnp.float32)
    out_ref[...] = jnp.maximum(o + b3_ref[...], 0.0)


def kernel(month, day, weekday, hour, month_table, day_table, weekday_table,
           hour_table, W1, b1, W2, b2, W3, b3):
    i32 = jnp.int32
    f32 = jnp.float32
    m = month.astype(i32)
    d = day.astype(i32)
    w = weekday.astype(i32)
    h = hour.astype(i32)
    tcat = jnp.concatenate(
        [month_table, day_table, weekday_table, hour_table,
         jnp.zeros((52, 128), f32)], axis=0)
    b1r = b1.reshape(1, 64)

    fold = pl.pallas_call(
        _fold_body,
        out_shape=jax.ShapeDtypeStruct((128, 64), f32),
    )(tcat, W1, b1r)
    fold = fold.reshape(128 * 64)

    mesh = plsc.VectorSubcoreMesh(core_axis_name="c", subcore_axis_name="s")
    h1t = pl.kernel(
        _gather_body,
        out_type=jax.ShapeDtypeStruct((64, _B), f32),
        mesh=mesh,
        compiler_params=pltpu.CompilerParams(needs_layout_passes=False),
        scratch_types=[
            pltpu.VMEM((128 * 64,), f32),
            pltpu.VMEM((_BPW,), i32),
            pltpu.VMEM((_BPW,), i32),
            pltpu.VMEM((_BPW,), i32),
            pltpu.VMEM((_BPW,), i32),
            pltpu.VMEM((64, _BPW), f32),
        ],
    )(fold, m, d, w, h)

    b2c = b2.reshape(32, 1)
    b3c = b3.reshape(1, 1)
    full = lambda s: pl.BlockSpec(s, lambda i: tuple(0 for _ in s))
    out = pl.pallas_call(
        _mlp_body,
        grid=(_NSTEPC,),
        in_specs=[pl.BlockSpec((64, _RC), lambda i: (0, i)),
                  full((64, 32)), full((32, 1)), full((32, 1)), full((1, 1))],
        out_specs=pl.BlockSpec((1, _RC), lambda i: (0, i)),
        out_shape=jax.ShapeDtypeStruct((1, _B), f32),
    )(h1t, W2, b2c, W3, b3c)
    return out.reshape(_B, 1)


# speedup vs baseline: 1.1786x; 1.1786x over previous
import functools
import jax
import jax.numpy as jnp
from jax import lax
from jax.experimental import pallas as pl
from jax.experimental.pallas import tpu as pltpu
from jax.experimental.pallas import tpu_sc as plsc

_B = 16384
_NC = 2
_NS = 16
_NW = _NC * _NS
_BPW = _B // _NW
_NG = _BPW // 16
_RC = 4096
_NSTEPC = _B // _RC


def _fold_body(tcat_ref, w1_ref, b1_ref, out_ref):
    f = jnp.dot(tcat_ref[...], w1_ref[...], preferred_element_type=jnp.float32,
                precision=jax.lax.Precision.HIGHEST)
    out_ref[...] = f + 0.25 * b1_ref[...]


def _gather_body(f_hbm, m_hbm, d_hbm, w_hbm, h_hbm, out_hbm,
                 f_v, mi_v, di_v, wi_v, hi_v, h1_v):
    wid = lax.axis_index("s") * _NC + lax.axis_index("c")
    base = wid * _BPW
    pltpu.sync_copy(f_hbm, f_v)
    pltpu.sync_copy(m_hbm.at[pl.ds(base, _BPW)], mi_v)
    pltpu.sync_copy(d_hbm.at[pl.ds(base, _BPW)], di_v)
    pltpu.sync_copy(w_hbm.at[pl.ds(base, _BPW)], wi_v)
    pltpu.sync_copy(h_hbm.at[pl.ds(base, _BPW)], hi_v)

    @plsc.parallel_loop(0, _NG)
    def g_body(g):
        o = g * 16
        mi = mi_v[pl.ds(o, 16)] * 64
        di = (di_v[pl.ds(o, 16)] + 13) * 64
        wi = (wi_v[pl.ds(o, 16)] + 45) * 64
        hi = (hi_v[pl.ds(o, 16)] + 52) * 64
        for k in range(64):
            e = (plsc.load_gather(f_v, [mi + k])
                 + plsc.load_gather(f_v, [di + k])
                 + plsc.load_gather(f_v, [wi + k])
                 + plsc.load_gather(f_v, [hi + k]))
            h1_v[k, pl.ds(o, 16)] = jnp.maximum(e, 0.0)
    pltpu.sync_copy(h1_v, out_hbm.at[:, pl.ds(base, _BPW)])


def _mlp_body(h1_ref, w2_ref, b2_ref, w3_ref, b3_ref, out_ref):
    c00 = (((0,), (0,)), ((), ()))
    h1 = h1_ref[...]
    h2 = jax.lax.dot_general(w2_ref[...], h1, c00,
                             preferred_element_type=jnp.float32)
    h2 = jnp.maximum(h2 + b2_ref[...], 0.0)
    o = jax.lax.dot_general(w3_ref[...], h2, c00,
                            preferred_element_type=jnp.float32)
    out_ref[...] = jnp.maximum(o + b3_ref[...], 0.0)


def kernel(month, day, weekday, hour, month_table, day_table, weekday_table,
           hour_table, W1, b1, W2, b2, W3, b3):
    i32 = jnp.int32
    f32 = jnp.float32
    m = month.astype(i32)
    d = day.astype(i32)
    w = weekday.astype(i32)
    h = hour.astype(i32)
    tcat = jnp.concatenate(
        [month_table, day_table, weekday_table, hour_table,
         jnp.zeros((52, 128), f32)], axis=0)
    b1r = b1.reshape(1, 64)

    fold = pl.pallas_call(
        _fold_body,
        out_shape=jax.ShapeDtypeStruct((128, 64), f32),
    )(tcat, W1, b1r)
    fold = fold.reshape(128 * 64)

    mesh = plsc.VectorSubcoreMesh(core_axis_name="c", subcore_axis_name="s")
    h1t = pl.kernel(
        _gather_body,
        out_type=jax.ShapeDtypeStruct((64, _B), f32),
        mesh=mesh,
        compiler_params=pltpu.CompilerParams(needs_layout_passes=False),
        scratch_types=[
            pltpu.VMEM((128 * 64,), f32),
            pltpu.VMEM((_BPW,), i32),
            pltpu.VMEM((_BPW,), i32),
            pltpu.VMEM((_BPW,), i32),
            pltpu.VMEM((_BPW,), i32),
            pltpu.VMEM((64, _BPW), f32),
        ],
    )(fold, m, d, w, h)

    b2c = b2.reshape(32, 1)
    b3c = b3.reshape(1, 1)
    full = lambda s: pl.BlockSpec(s, lambda i: tuple(0 for _ in s))
    out = pl.pallas_call(
        _mlp_body,
        grid=(_NSTEPC,),
        in_specs=[pl.BlockSpec((64, _RC), lambda i: (0, i)),
                  full((64, 32)), full((32, 1)), full((32, 1)), full((1, 1))],
        out_specs=pl.BlockSpec((1, _RC), lambda i: (0, i)),
        out_shape=jax.ShapeDtypeStruct((1, _B), f32),
    )(h1t, W2, b2c, W3, b3c)
    return out.reshape(_B, 1)


# speedup vs baseline: 2.6356x; 2.2362x over previous
import functools
import jax
import jax.numpy as jnp
from jax import lax
from jax.experimental import pallas as pl
from jax.experimental.pallas import tpu as pltpu
from jax.experimental.pallas import tpu_sc as plsc

_B = 16384
_NC = 2
_NS = 16
_NW = _NC * _NS
_BPW = _B // _NW
_NG = _BPW // 16
_RC = 4096
_NSTEPC = _B // _RC
_FS = 65


def _fold_body(tcat_ref, w1_ref, b1_ref, out_ref):
    f = jnp.dot(tcat_ref[...], w1_ref[...], preferred_element_type=jnp.float32,
                precision=jax.lax.Precision.HIGHEST)
    out_ref[...] = f + 0.25 * b1_ref[...]


def _gather_body(f_hbm, m_hbm, d_hbm, w_hbm, h_hbm, out_hbm,
                 f_v, mi_v, di_v, wi_v, hi_v, h1_v):
    wid = lax.axis_index("s") * _NC + lax.axis_index("c")
    base = wid * _BPW
    pltpu.sync_copy(f_hbm, f_v)
    pltpu.sync_copy(m_hbm.at[pl.ds(base, _BPW)], mi_v)
    pltpu.sync_copy(d_hbm.at[pl.ds(base, _BPW)], di_v)
    pltpu.sync_copy(w_hbm.at[pl.ds(base, _BPW)], wi_v)
    pltpu.sync_copy(h_hbm.at[pl.ds(base, _BPW)], hi_v)

    @plsc.parallel_loop(0, _NG)
    def g_body(g):
        o = g * 16
        mi = mi_v[pl.ds(o, 16)] * _FS
        di = (di_v[pl.ds(o, 16)] + 13) * _FS
        wi = (wi_v[pl.ds(o, 16)] + 45) * _FS
        hi = (hi_v[pl.ds(o, 16)] + 52) * _FS
        for k in range(64):
            e = ((plsc.load_gather(f_v, [mi + k])
                  + plsc.load_gather(f_v, [di + k]))
                 + (plsc.load_gather(f_v, [wi + k])
                    + plsc.load_gather(f_v, [hi + k])))
            h1_v[k, pl.ds(o, 16)] = jnp.maximum(e, 0.0)
    pltpu.sync_copy(h1_v, out_hbm.at[:, pl.ds(base, _BPW)])


def _mlp_body(h1_ref, w2_ref, b2_ref, w3_ref, b3_ref, out_ref):
    c00 = (((0,), (0,)), ((), ()))
    h1 = h1_ref[...]
    h2 = jax.lax.dot_general(w2_ref[...], h1, c00,
                             preferred_element_type=jnp.float32)
    h2 = jnp.maximum(h2 + b2_ref[...], 0.0)
    o = jax.lax.dot_general(w3_ref[...], h2, c00,
                            preferred_element_type=jnp.float32)
    out_ref[...] = jnp.maximum(o + b3_ref[...], 0.0)


def kernel(month, day, weekday, hour, month_table, day_table, weekday_table,
           hour_table, W1, b1, W2, b2, W3, b3):
    i32 = jnp.int32
    f32 = jnp.float32
    m = month.astype(i32)
    d = day.astype(i32)
    w = weekday.astype(i32)
    h = hour.astype(i32)
    tcat = jnp.concatenate(
        [month_table, day_table, weekday_table, hour_table,
         jnp.zeros((52, 128), f32)], axis=0)
    b1r = b1.reshape(1, 64)

    fold = pl.pallas_call(
        _fold_body,
        out_shape=jax.ShapeDtypeStruct((128, 64), f32),
    )(tcat, W1, b1r)
    fold = jnp.pad(fold, ((0, 0), (0, _FS - 64))).reshape(128 * _FS)

    mesh = plsc.VectorSubcoreMesh(core_axis_name="c", subcore_axis_name="s")
    h1t = pl.kernel(
        _gather_body,
        out_type=jax.ShapeDtypeStruct((64, _B), f32),
        mesh=mesh,
        compiler_params=pltpu.CompilerParams(needs_layout_passes=False),
        scratch_types=[
            pltpu.VMEM((128 * _FS,), f32),
            pltpu.VMEM((_BPW,), i32),
            pltpu.VMEM((_BPW,), i32),
            pltpu.VMEM((_BPW,), i32),
            pltpu.VMEM((_BPW,), i32),
            pltpu.VMEM((64, _BPW), f32),
        ],
    )(fold, m, d, w, h)

    b2c = b2.reshape(32, 1)
    b3c = b3.reshape(1, 1)
    full = lambda s: pl.BlockSpec(s, lambda i: tuple(0 for _ in s))
    out = pl.pallas_call(
        _mlp_body,
        grid=(_NSTEPC,),
        in_specs=[pl.BlockSpec((64, _RC), lambda i: (0, i)),
                  full((64, 32)), full((32, 1)), full((32, 1)), full((1, 1))],
        out_specs=pl.BlockSpec((1, _RC), lambda i: (0, i)),
        out_shape=jax.ShapeDtypeStruct((1, _B), f32),
    )(h1t, W2, b2c, W3, b3c)
    return out.reshape(_B, 1)


# speedup vs baseline: 4.5402x; 1.7226x over previous
import jax
import jax.numpy as jnp
from jax.experimental import pallas as pl

_B = 16384
_R = 2048
_NSTEP = _B // _R


def _body(m_ref, d_ref, w_ref, h_ref, tcat_ref, w1_ref, b1_ref, w2_ref,
          b2_ref, w3_ref, b3_ref, out_ref):
    f32 = jnp.float32
    m = m_ref[0]
    d = d_ref[0]
    w = w_ref[0]
    h = h_ref[0]
    iota = jax.lax.broadcasted_iota(jnp.int32, (128, _R), 0)
    hot = ((iota == m) | (iota == d + 13) | (iota == w + 45)
           | (iota == h + 52))
    mh = jnp.where(hot, f32(1.0), f32(0.0))

    c00 = (((0,), (0,)), ((), ()))
    comb = jax.lax.dot_general(tcat_ref[...], mh, c00,
                               preferred_element_type=f32,
                               precision=jax.lax.Precision.HIGHEST)
    h1 = jax.lax.dot_general(w1_ref[...], comb, c00,
                             preferred_element_type=f32)
    h1 = jnp.maximum(h1 + b1_ref[...], 0.0)
    h2 = jax.lax.dot_general(w2_ref[...], h1, c00,
                             preferred_element_type=f32)
    h2 = jnp.maximum(h2 + b2_ref[...], 0.0)
    o = jax.lax.dot_general(w3_ref[...], h2, c00,
                            preferred_element_type=f32)
    o = jnp.maximum(o + b3_ref[...], 0.0)
    out_ref[...] = o.reshape(1, 1, _R)


def kernel(month, day, weekday, hour, month_table, day_table, weekday_table,
           hour_table, W1, b1, W2, b2, W3, b3):
    i32 = jnp.int32
    f32 = jnp.float32
    m = month.astype(i32).reshape(_NSTEP, 1, _R)
    d = day.astype(i32).reshape(_NSTEP, 1, _R)
    w = weekday.astype(i32).reshape(_NSTEP, 1, _R)
    h = hour.astype(i32).reshape(_NSTEP, 1, _R)
    tcat = jnp.concatenate(
        [month_table, day_table, weekday_table, hour_table,
         jnp.zeros((52, 128), f32)], axis=0)
    b1c = b1.reshape(64, 1)
    b2c = b2.reshape(32, 1)
    b3c = b3.reshape(1, 1)

    idx_spec = pl.BlockSpec((1, 1, _R), lambda i: (i, 0, 0))
    full = lambda s: pl.BlockSpec(s, lambda i: tuple(0 for _ in s))
    out = pl.pallas_call(
        _body,
        grid=(_NSTEP,),
        in_specs=[idx_spec, idx_spec, idx_spec, idx_spec,
                  full((128, 128)), full((128, 64)), full((64, 1)),
                  full((64, 32)), full((32, 1)), full((32, 1)),
                  full((1, 1))],
        out_specs=pl.BlockSpec((1, 1, _R), lambda i: (i, 0, 0)),
        out_shape=jax.ShapeDtypeStruct((_NSTEP, 1, _R), f32),
    )(m, d, w, h, tcat, W1, b1c, W2, b2c, W3, b3c)
    return out.reshape(_B, 1)


# speedup vs baseline: 5.1476x; 1.1338x over previous
import jax
import jax.numpy as jnp
from jax.experimental import pallas as pl

_B = 16384
_R = 2048
_NSTEP = _B // _R


def _body(m_ref, d_ref, w_ref, h_ref, tcat_ref, w1_ref, b1_ref, w2_ref,
          b2_ref, w3_ref, b3_ref, out_ref):
    f32 = jnp.float32
    m = m_ref[0]
    d = d_ref[0]
    w = w_ref[0]
    h = h_ref[0]
    iota = jax.lax.broadcasted_iota(jnp.int32, (128, _R), 0)
    hot = ((iota == m) | (iota == d + 13) | (iota == w + 45)
           | (iota == h + 52))
    mh = jnp.where(hot, f32(1.0), f32(0.0))

    c00 = (((0,), (0,)), ((), ()))
    tcat = tcat_ref[...]
    t_hi = tcat.astype(jnp.bfloat16).astype(f32)
    t_lo = tcat - t_hi
    comb = (jax.lax.dot_general(t_hi, mh, c00, preferred_element_type=f32)
            + jax.lax.dot_general(t_lo, mh, c00,
                                  preferred_element_type=f32))
    h1 = jax.lax.dot_general(w1_ref[...], comb, c00,
                             preferred_element_type=f32)
    h1 = jnp.maximum(h1 + b1_ref[...], 0.0)
    h2 = jax.lax.dot_general(w2_ref[...], h1, c00,
                             preferred_element_type=f32)
    h2 = jnp.maximum(h2 + b2_ref[...], 0.0)
    o = jax.lax.dot_general(w3_ref[...], h2, c00,
                            preferred_element_type=f32)
    o = jnp.maximum(o + b3_ref[...], 0.0)
    out_ref[...] = o.reshape(1, 1, _R)


def kernel(month, day, weekday, hour, month_table, day_table, weekday_table,
           hour_table, W1, b1, W2, b2, W3, b3):
    i32 = jnp.int32
    f32 = jnp.float32
    m = month.astype(i32).reshape(_NSTEP, 1, _R)
    d = day.astype(i32).reshape(_NSTEP, 1, _R)
    w = weekday.astype(i32).reshape(_NSTEP, 1, _R)
    h = hour.astype(i32).reshape(_NSTEP, 1, _R)
    tcat = jnp.concatenate(
        [month_table, day_table, weekday_table, hour_table,
         jnp.zeros((52, 128), f32)], axis=0)
    b1c = b1.reshape(64, 1)
    b2c = b2.reshape(32, 1)
    b3c = b3.reshape(1, 1)

    idx_spec = pl.BlockSpec((1, 1, _R), lambda i: (i, 0, 0))
    full = lambda s: pl.BlockSpec(s, lambda i: tuple(0 for _ in s))
    out = pl.pallas_call(
        _body,
        grid=(_NSTEP,),
        in_specs=[idx_spec, idx_spec, idx_spec, idx_spec,
                  full((128, 128)), full((128, 64)), full((64, 1)),
                  full((64, 32)), full((32, 1)), full((32, 1)),
                  full((1, 1))],
        out_specs=pl.BlockSpec((1, 1, _R), lambda i: (i, 0, 0)),
        out_shape=jax.ShapeDtypeStruct((_NSTEP, 1, _R), f32),
    )(m, d, w, h, tcat, W1, b1c, W2, b2c, W3, b3c)
    return out.reshape(_B, 1)


# speedup vs baseline: 5.7517x; 1.1173x over previous
import jax
import jax.numpy as jnp
from jax.experimental import pallas as pl

_B = 16384
_R = 4096
_NSTEP = _B // _R


def _body(m_ref, d_ref, w_ref, h_ref, tcat_ref, w1_ref, b1_ref, w2_ref,
          b2_ref, w3_ref, b3_ref, out_ref):
    f32 = jnp.float32
    m = m_ref[0]
    d = d_ref[0]
    w = w_ref[0]
    h = h_ref[0]
    iota = jax.lax.broadcasted_iota(jnp.int32, (128, _R), 0)
    hot = ((iota == m) | (iota == d + 13) | (iota == w + 45)
           | (iota == h + 52))
    mh = jnp.where(hot, f32(1.0), f32(0.0))

    c00 = (((0,), (0,)), ((), ()))
    tcat = tcat_ref[...]
    t_hi = tcat.astype(jnp.bfloat16).astype(f32)
    t_lo = tcat - t_hi
    comb = (jax.lax.dot_general(t_hi, mh, c00, preferred_element_type=f32)
            + jax.lax.dot_general(t_lo, mh, c00,
                                  preferred_element_type=f32))
    h1 = jax.lax.dot_general(w1_ref[...], comb, c00,
                             preferred_element_type=f32)
    h1 = jnp.maximum(h1 + b1_ref[...], 0.0)
    h2 = jax.lax.dot_general(w2_ref[...], h1, c00,
                             preferred_element_type=f32)
    h2 = jnp.maximum(h2 + b2_ref[...], 0.0)
    o = jax.lax.dot_general(w3_ref[...], h2, c00,
                            preferred_element_type=f32)
    o = jnp.maximum(o + b3_ref[...], 0.0)
    out_ref[...] = o.reshape(1, 1, _R)


def kernel(month, day, weekday, hour, month_table, day_table, weekday_table,
           hour_table, W1, b1, W2, b2, W3, b3):
    i32 = jnp.int32
    f32 = jnp.float32
    m = month.astype(i32).reshape(_NSTEP, 1, _R)
    d = day.astype(i32).reshape(_NSTEP, 1, _R)
    w = weekday.astype(i32).reshape(_NSTEP, 1, _R)
    h = hour.astype(i32).reshape(_NSTEP, 1, _R)
    tcat = jnp.concatenate(
        [month_table, day_table, weekday_table, hour_table,
         jnp.zeros((52, 128), f32)], axis=0)
    b1c = b1.reshape(64, 1)
    b2c = b2.reshape(32, 1)
    b3c = b3.reshape(1, 1)

    idx_spec = pl.BlockSpec((1, 1, _R), lambda i: (i, 0, 0))
    full = lambda s: pl.BlockSpec(s, lambda i: tuple(0 for _ in s))
    out = pl.pallas_call(
        _body,
        grid=(_NSTEP,),
        in_specs=[idx_spec, idx_spec, idx_spec, idx_spec,
                  full((128, 128)), full((128, 64)), full((64, 1)),
                  full((64, 32)), full((32, 1)), full((32, 1)),
                  full((1, 1))],
        out_specs=pl.BlockSpec((1, 1, _R), lambda i: (i, 0, 0)),
        out_shape=jax.ShapeDtypeStruct((_NSTEP, 1, _R), f32),
    )(m, d, w, h, tcat, W1, b1c, W2, b2c, W3, b3c)
    return out.reshape(_B, 1)


# speedup vs baseline: 6.0159x; 1.0459x over previous
import jax
import jax.numpy as jnp
from jax.experimental import pallas as pl

_B = 16384
_R = 8192
_NSTEP = _B // _R


def _body(m_ref, d_ref, w_ref, h_ref, tcat_ref, w1_ref, b1_ref, w2_ref,
          b2_ref, w3_ref, b3_ref, out_ref):
    f32 = jnp.float32
    m = m_ref[0]
    d = d_ref[0]
    w = w_ref[0]
    h = h_ref[0]
    iota = jax.lax.broadcasted_iota(jnp.int32, (128, _R), 0)
    hot = ((iota == m) | (iota == d + 13) | (iota == w + 45)
           | (iota == h + 52))
    mh = jnp.where(hot, f32(1.0), f32(0.0))

    c00 = (((0,), (0,)), ((), ()))
    tcat = tcat_ref[...]
    t_hi = tcat.astype(jnp.bfloat16).astype(f32)
    t_lo = tcat - t_hi
    comb = (jax.lax.dot_general(t_hi, mh, c00, preferred_element_type=f32)
            + jax.lax.dot_general(t_lo, mh, c00,
                                  preferred_element_type=f32))
    h1 = jax.lax.dot_general(w1_ref[...], comb, c00,
                             preferred_element_type=f32)
    h1 = jnp.maximum(h1 + b1_ref[...], 0.0)
    h2 = jax.lax.dot_general(w2_ref[...], h1, c00,
                             preferred_element_type=f32)
    h2 = jnp.maximum(h2 + b2_ref[...], 0.0)
    o = jax.lax.dot_general(w3_ref[...], h2, c00,
                            preferred_element_type=f32)
    o = jnp.maximum(o + b3_ref[...], 0.0)
    out_ref[...] = o.reshape(1, 1, _R)


def kernel(month, day, weekday, hour, month_table, day_table, weekday_table,
           hour_table, W1, b1, W2, b2, W3, b3):
    i32 = jnp.int32
    f32 = jnp.float32
    m = month.astype(i32).reshape(_NSTEP, 1, _R)
    d = day.astype(i32).reshape(_NSTEP, 1, _R)
    w = weekday.astype(i32).reshape(_NSTEP, 1, _R)
    h = hour.astype(i32).reshape(_NSTEP, 1, _R)
    tcat = jnp.concatenate(
        [month_table, day_table, weekday_table, hour_table,
         jnp.zeros((52, 128), f32)], axis=0)
    b1c = b1.reshape(64, 1)
    b2c = b2.reshape(32, 1)
    b3c = b3.reshape(1, 1)

    idx_spec = pl.BlockSpec((1, 1, _R), lambda i: (i, 0, 0))
    full = lambda s: pl.BlockSpec(s, lambda i: tuple(0 for _ in s))
    out = pl.pallas_call(
        _body,
        grid=(_NSTEP,),
        in_specs=[idx_spec, idx_spec, idx_spec, idx_spec,
                  full((128, 128)), full((128, 64)), full((64, 1)),
                  full((64, 32)), full((32, 1)), full((32, 1)),
                  full((1, 1))],
        out_specs=pl.BlockSpec((1, 1, _R), lambda i: (i, 0, 0)),
        out_shape=jax.ShapeDtypeStruct((_NSTEP, 1, _R), f32),
    )(m, d, w, h, tcat, W1, b1c, W2, b2c, W3, b3c)
    return out.reshape(_B, 1)
